# baseline (speedup 1.0000x reference)
import jax
import jax.numpy as jnp
from jax import lax
from jax.experimental import pallas as pl
from jax.experimental.pallas import tpu as pltpu
from jax.experimental.pallas import tpu_sc as plsc

_N = 16384
_OUT_D = 129
_GATHER_D = 128
_NC = 2
_NS = 16
_NW = _NC * _NS
_RPW = _N // _NW
_GCHUNK = 128
_NG = _RPW // _GCHUNK
_N_TYPES = 16


def _body(x_hbm, ext_hbm, out_hbm, xs, idx2, gbuf, zbuf, sin, sg, so1, so2):
    wid = lax.axis_index("s") * _NC + lax.axis_index("c")
    base = wid * _RPW

    cin = pltpu.async_copy(
        x_hbm.at[pl.ds(base, _RPW), pl.ds(0, 2)], xs, sin)

    lanes = lax.iota(jnp.int32, 16)
    col0 = jnp.zeros((16,), jnp.int32)
    col1 = jnp.ones((16,), jnp.int32)
    zero16 = jnp.zeros((16,), jnp.int32)
    cap16 = jnp.full((16,), _N_TYPES - 1, jnp.int32)
    zf16 = jnp.zeros((16,), jnp.float32)

    for g in range(_RPW // 16):
        plsc.store_scatter(zbuf, [lanes + g * 16, col0], zf16)

    cin.wait()
    for j in range(_NG):
        for g in range(_GCHUNK // 16):
            rows = lanes + (j * _GCHUNK + g * 16)
            x1 = plsc.load_gather(xs, [rows, col1])
            idx = jnp.minimum(jnp.maximum(x1.astype(jnp.int32), zero16), cap16)
            idx2[j, pl.ds(g * 16, 16)] = idx

    gathers = [
        pltpu.async_copy(
            ext_hbm.at[idx2.at[j]],
            gbuf.at[pl.ds(j * _GCHUNK, _GCHUNK)],
            sg,
        )
        for j in range(_NG)
    ]
    for c in gathers:
        c.wait()

    for g in range(_RPW // 16):
        rows = lanes + g * 16
        pre = plsc.load_gather(xs, [rows, col0])
        plsc.store_scatter(gbuf, [rows, col0], pre)

    c1 = pltpu.async_copy(
        gbuf, out_hbm.at[pl.ds(base, _RPW), pl.ds(0, _GATHER_D)], so1)
    c2 = pltpu.async_copy(
        zbuf, out_hbm.at[pl.ds(base, _RPW), pl.ds(_GATHER_D, 1)], so2)
    c1.wait()
    c2.wait()


@jax.jit
def _sc_embed(X, ext_table):
    mesh = plsc.VectorSubcoreMesh(core_axis_name="c", subcore_axis_name="s")
    return pl.kernel(
        _body,
        out_type=jax.ShapeDtypeStruct((_N, _OUT_D), jnp.float32),
        mesh=mesh,
        scratch_types=[
            pltpu.VMEM((_RPW, 2), jnp.float32),
            pltpu.VMEM((_NG, _GCHUNK), jnp.int32),
            pltpu.VMEM((_RPW, _GATHER_D), jnp.float32),
            pltpu.VMEM((_RPW, 1), jnp.float32),
            pltpu.SemaphoreType.DMA,
            pltpu.SemaphoreType.DMA,
            pltpu.SemaphoreType.DMA,
            pltpu.SemaphoreType.DMA,
        ],
        compiler_params=pltpu.CompilerParams(
            use_tc_tiling_on_sc=False, needs_layout_passes=False
        ),
    )(X, ext_table)


def kernel(X, type_table):
    ext = jnp.zeros((_N_TYPES, _GATHER_D), jnp.float32)
    ext = lax.dynamic_update_slice(ext, type_table, (0, 1))
    return _sc_embed(X, ext)


# speedup vs baseline: 6.4740x; 6.4740x over previous
import jax
import jax.numpy as jnp
from jax import lax
from jax.experimental import pallas as pl
from jax.experimental.pallas import tpu as pltpu
from jax.experimental.pallas import tpu_sc as plsc

_N = 16384
_OUT_D = 129
_GATHER_D = 128
_NC = 2
_NS = 16
_NW = _NC * _NS
_RPW = _N // _NW
_GCHUNK = 128
_NG = _RPW // _GCHUNK
_N_TYPES = 16


def _body(x_hbm, ext_hbm, out_hbm, sh, xs, idx2, gbuf, zbuf, sin, sg, so1, so2):
    sid = lax.axis_index("s")
    wid = sid * _NC + lax.axis_index("c")
    base = wid * _RPW

    cin = pltpu.async_copy(
        x_hbm.at[pl.ds(base, _RPW), pl.ds(0, 2)], xs, sin)

    @pl.when(sid == 0)
    def _():
        pltpu.sync_copy(ext_hbm, sh)

    lanes = lax.iota(jnp.int32, 16)
    col0 = jnp.zeros((16,), jnp.int32)
    col1 = jnp.ones((16,), jnp.int32)
    zero16 = jnp.zeros((16,), jnp.int32)
    cap16 = jnp.full((16,), _N_TYPES - 1, jnp.int32)
    zf16 = jnp.zeros((16,), jnp.float32)

    for g in range(_RPW // 16):
        plsc.store_scatter(zbuf, [lanes + g * 16, col0], zf16)

    plsc.subcore_barrier()

    cin.wait()
    for j in range(_NG):
        for g in range(_GCHUNK // 16):
            rows = lanes + (j * _GCHUNK + g * 16)
            x1 = plsc.load_gather(xs, [rows, col1])
            idx = jnp.minimum(jnp.maximum(x1.astype(jnp.int32), zero16), cap16)
            idx2[j, pl.ds(g * 16, 16)] = idx

    gathers = [
        pltpu.async_copy(
            sh.at[idx2.at[j]],
            gbuf.at[pl.ds(j * _GCHUNK, _GCHUNK)],
            sg,
        )
        for j in range(_NG)
    ]
    for c in gathers:
        c.wait()

    for g in range(_RPW // 16):
        rows = lanes + g * 16
        pre = plsc.load_gather(xs, [rows, col0])
        plsc.store_scatter(gbuf, [rows, col0], pre)

    c1 = pltpu.async_copy(
        gbuf, out_hbm.at[pl.ds(base, _RPW), pl.ds(0, _GATHER_D)], so1)
    c2 = pltpu.async_copy(
        zbuf, out_hbm.at[pl.ds(base, _RPW), pl.ds(_GATHER_D, 1)], so2)
    c1.wait()
    c2.wait()


@jax.jit
def _sc_embed(X, ext_table):
    mesh = plsc.VectorSubcoreMesh(core_axis_name="c", subcore_axis_name="s")
    return pl.kernel(
        _body,
        out_type=jax.ShapeDtypeStruct((_N, _OUT_D), jnp.float32),
        mesh=mesh,
        scratch_types=[
            pltpu.VMEM_SHARED((_N_TYPES, _GATHER_D), jnp.float32),
            pltpu.VMEM((_RPW, 2), jnp.float32),
            pltpu.VMEM((_NG, _GCHUNK), jnp.int32),
            pltpu.VMEM((_RPW, _GATHER_D), jnp.float32),
            pltpu.VMEM((_RPW, 1), jnp.float32),
            pltpu.SemaphoreType.DMA,
            pltpu.SemaphoreType.DMA,
            pltpu.SemaphoreType.DMA,
            pltpu.SemaphoreType.DMA,
        ],
        compiler_params=pltpu.CompilerParams(
            use_tc_tiling_on_sc=False, needs_layout_passes=False
        ),
    )(X, ext_table)


def kernel(X, type_table):
    ext = jnp.zeros((_N_TYPES, _GATHER_D), jnp.float32)
    ext = lax.dynamic_update_slice(ext, type_table, (0, 1))
    return _sc_embed(X, ext)


# speedup vs baseline: 6.4945x; 1.0032x over previous
import jax
import jax.numpy as jnp
from jax import lax
from jax.experimental import pallas as pl
from jax.experimental.pallas import tpu as pltpu
from jax.experimental.pallas import tpu_sc as plsc

_N = 16384
_OUT_D = 129
_GATHER_D = 128
_NC = 2
_NS = 16
_NW = _NC * _NS
_RPW = _N // _NW
_GCHUNK = 128
_NG = _RPW // _GCHUNK
_N_TYPES = 16


def _body(x_hbm, ext_hbm, out_hbm, sh, xs, idx2, gbuf, zbuf,
          sin, sga, sgb, so1, so2):
    sid = lax.axis_index("s")
    wid = sid * _NC + lax.axis_index("c")
    base = wid * _RPW

    cin = pltpu.async_copy(
        x_hbm.at[pl.ds(base, _RPW), pl.ds(0, 2)], xs, sin)

    @pl.when(sid == 0)
    def _():
        pltpu.sync_copy(ext_hbm, sh)

    lanes = lax.iota(jnp.int32, 16)
    col0 = jnp.zeros((16,), jnp.int32)
    col1 = jnp.ones((16,), jnp.int32)
    zero16 = jnp.zeros((16,), jnp.int32)
    cap16 = jnp.full((16,), _N_TYPES - 1, jnp.int32)
    zf16 = jnp.zeros((16,), jnp.float32)

    for g in range(_RPW // 16):
        plsc.store_scatter(zbuf, [lanes + g * 16, col0], zf16)
    c2 = pltpu.async_copy(
        zbuf, out_hbm.at[pl.ds(base, _RPW), pl.ds(_GATHER_D, 1)], so2)

    plsc.subcore_barrier()

    cin.wait()
    for j in range(_NG):
        for g in range(_GCHUNK // 16):
            rows = lanes + (j * _GCHUNK + g * 16)
            x1 = plsc.load_gather(xs, [rows, col1])
            idx = jnp.minimum(jnp.maximum(x1.astype(jnp.int32), zero16), cap16)
            idx2[j, pl.ds(g * 16, 16)] = idx

    half = _RPW // 2
    sems = (sga, sgb)
    gathers = [
        pltpu.async_copy(
            sh.at[idx2.at[j]],
            gbuf.at[pl.ds(j * _GCHUNK, _GCHUNK)],
            sems[j // (_NG // 2)],
        )
        for j in range(_NG)
    ]

    outs = []
    for h in range(2):
        for c in gathers[h * (_NG // 2):(h + 1) * (_NG // 2)]:
            c.wait()
        for g in range(half // 16):
            rows = lanes + (h * half + g * 16)
            pre = plsc.load_gather(xs, [rows, col0])
            plsc.store_scatter(gbuf, [rows, col0], pre)
        outs.append(pltpu.async_copy(
            gbuf.at[pl.ds(h * half, half)],
            out_hbm.at[pl.ds(base + h * half, half), pl.ds(0, _GATHER_D)],
            so1,
        ))
    for c in outs:
        c.wait()
    c2.wait()


@jax.jit
def _sc_embed(X, ext_table):
    mesh = plsc.VectorSubcoreMesh(core_axis_name="c", subcore_axis_name="s")
    return pl.kernel(
        _body,
        out_type=jax.ShapeDtypeStruct((_N, _OUT_D), jnp.float32),
        mesh=mesh,
        scratch_types=[
            pltpu.VMEM_SHARED((_N_TYPES, _GATHER_D), jnp.float32),
            pltpu.VMEM((_RPW, 2), jnp.float32),
            pltpu.VMEM((_NG, _GCHUNK), jnp.int32),
            pltpu.VMEM((_RPW, _GATHER_D), jnp.float32),
            pltpu.VMEM((_RPW, 1), jnp.float32),
            pltpu.SemaphoreType.DMA,
            pltpu.SemaphoreType.DMA,
            pltpu.SemaphoreType.DMA,
            pltpu.SemaphoreType.DMA,
            pltpu.SemaphoreType.DMA,
        ],
        compiler_params=pltpu.CompilerParams(
            use_tc_tiling_on_sc=False, needs_layout_passes=False
        ),
    )(X, ext_table)


def kernel(X, type_table):
    ext = jnp.zeros((_N_TYPES, _GATHER_D), jnp.float32)
    ext = lax.dynamic_update_slice(ext, type_table, (0, 1))
    return _sc_embed(X, ext)
